# Initial kernel scaffold; baseline (speedup 1.0000x reference)
import jax
import jax.numpy as jnp
from jax import lax
from jax.experimental import pallas as pl
from jax.experimental.pallas import tpu as pltpu
from jax.experimental.pallas import tpu_sc as plsc

N = 10000
E = 160000
FH = 128
NTILES = 16
EPT = E // NTILES
B = 128
NFULL = EPT // B
TAIL = EPT - NFULL * B
ROWS_A = 624
ROWS_B = N - 15 * ROWS_A


def _sc_agg_body(xa, xb, src_h, dst_h, zeros2d, zeros1d,
                 agg_a, agg_b, deg_out,
                 sidx, didx, sidx_t, didx_t, rows, rows_t, ones_v, acc_sh,
                 deg_sh, sem, *, with_deg):
    c = lax.axis_index("c")
    s = lax.axis_index("s")
    base_rows = s * ROWS_A

    @pl.when(s < 15)
    def _():
        pltpu.sync_copy(zeros2d.at[pl.ds(0, ROWS_A)],
                        acc_sh.at[pl.ds(base_rows, ROWS_A)])

    @pl.when(s == 15)
    def _():
        pltpu.sync_copy(zeros2d, acc_sh.at[pl.ds(15 * ROWS_A, ROWS_B)])

    if with_deg:
        @pl.when(c == 0)
        def _():
            @pl.when(s < 15)
            def _():
                pltpu.sync_copy(zeros1d.at[pl.ds(0, ROWS_A)],
                                deg_sh.at[pl.ds(base_rows, ROWS_A)])

            @pl.when(s == 15)
            def _():
                pltpu.sync_copy(zeros1d, deg_sh.at[pl.ds(15 * ROWS_A, ROWS_B)])

        for j in range(B // 16):
            ones_v[pl.ds(j * 16, 16)] = jnp.ones((16,), jnp.float32)

    plsc.subcore_barrier()

    def edge_loop(table, do_deg):
        def body(i, carry):
            base = pl.multiple_of(s * EPT + i * B, 8)
            pltpu.sync_copy(src_h.at[pl.ds(base, B)], sidx)
            pltpu.sync_copy(dst_h.at[pl.ds(base, B)], didx)
            pltpu.async_copy(table.at[sidx], rows, sem).wait()
            pltpu.sync_copy(rows, acc_sh.at[didx], add=True)
            if do_deg:
                pltpu.sync_copy(ones_v, deg_sh.at[didx], add=True)
            return carry

        lax.fori_loop(0, NFULL, body, 0)
        tbase = pl.multiple_of(s * EPT + NFULL * B, 8)
        pltpu.sync_copy(src_h.at[pl.ds(tbase, TAIL)], sidx_t)
        pltpu.sync_copy(dst_h.at[pl.ds(tbase, TAIL)], didx_t)
        pltpu.async_copy(table.at[sidx_t], rows_t, sem).wait()
        pltpu.sync_copy(rows_t, acc_sh.at[didx_t], add=True)
        if do_deg:
            pltpu.sync_copy(ones_v.at[pl.ds(0, TAIL)], deg_sh.at[didx_t],
                            add=True)

    @pl.when(c == 0)
    def _():
        edge_loop(xa, with_deg)

    @pl.when(c == 1)
    def _():
        edge_loop(xb, False)

    plsc.subcore_barrier()

    def copy_out(dst):
        @pl.when(s < 15)
        def _():
            pltpu.sync_copy(acc_sh.at[pl.ds(base_rows, ROWS_A)],
                            dst.at[pl.ds(base_rows, ROWS_A)])

        @pl.when(s == 15)
        def _():
            pltpu.sync_copy(acc_sh.at[pl.ds(15 * ROWS_A, ROWS_B)],
                            dst.at[pl.ds(15 * ROWS_A, ROWS_B)])

    @pl.when(c == 0)
    def _():
        copy_out(agg_a)

    @pl.when(c == 1)
    def _():
        copy_out(agg_b)

    if with_deg:
        @pl.when(c == 0)
        def _():
            @pl.when(s < 15)
            def _():
                pltpu.sync_copy(deg_sh.at[pl.ds(base_rows, ROWS_A)],
                                deg_out.at[pl.ds(base_rows, ROWS_A)])

            @pl.when(s == 15)
            def _():
                pltpu.sync_copy(deg_sh.at[pl.ds(15 * ROWS_A, ROWS_B)],
                                deg_out.at[pl.ds(15 * ROWS_A, ROWS_B)])


def _make_sc_agg(with_deg):
    out_type = [jax.ShapeDtypeStruct((N, FH), jnp.float32),
                jax.ShapeDtypeStruct((N, FH), jnp.float32)]
    if with_deg:
        out_type.append(jax.ShapeDtypeStruct((N,), jnp.float32))
    mesh = plsc.VectorSubcoreMesh(core_axis_name="c", subcore_axis_name="s")
    scratch = [
        pltpu.VMEM((B,), jnp.int32),
        pltpu.VMEM((B,), jnp.int32),
        pltpu.VMEM((TAIL,), jnp.int32),
        pltpu.VMEM((TAIL,), jnp.int32),
        pltpu.VMEM((B, FH), jnp.float32),
        pltpu.VMEM((TAIL, FH), jnp.float32),
        pltpu.VMEM((B,), jnp.float32),
        pltpu.VMEM_SHARED((N, FH), jnp.float32),
        pltpu.VMEM_SHARED((N,), jnp.float32),
        pltpu.SemaphoreType.DMA,
    ]

    def body(*refs):
        xa, xb, src_h, dst_h, zeros2d, zeros1d = refs[:6]
        rest = refs[6:]
        if with_deg:
            agg_a, agg_b, deg_out = rest[:3]
            scr = rest[3:]
        else:
            agg_a, agg_b = rest[:2]
            deg_out = None
            scr = rest[2:]
        _sc_agg_body(xa, xb, src_h, dst_h, zeros2d, zeros1d,
                     agg_a, agg_b, deg_out, *scr, with_deg=with_deg)

    return pl.kernel(body, out_type=out_type, mesh=mesh,
                     scratch_types=scratch)


_sc_agg_deg = _make_sc_agg(True)
_sc_agg = _make_sc_agg(False)

RB = 1000
GRID = N // RB


def _tc_layer1_body(agg_a, agg_b, xa, xb, deg, wla, wlb, wra, wrb, b1,
                    h1a, h1b):
    invd = 1.0 / jnp.maximum(deg[...], 1.0)
    h = (jnp.dot(agg_a[...] * invd, wla[...],
                 preferred_element_type=jnp.float32)
         + jnp.dot(agg_b[...] * invd, wlb[...],
                   preferred_element_type=jnp.float32)
         + jnp.dot(xa[...], wra[...], preferred_element_type=jnp.float32)
         + jnp.dot(xb[...], wrb[...], preferred_element_type=jnp.float32)
         + b1[...])
    h = jnp.maximum(h, 0.0)
    h1a[...] = h[:, :FH]
    h1b[...] = h[:, FH:]


def _tc_layer2_body(agg_a, agg_b, h1a, h1b, deg, wla, wlb, wra, wrb, b2,
                    wcls, bcls, wc, bn_scale, bn_shift, logists, out):
    invd = 1.0 / jnp.maximum(deg[...], 1.0)
    h = (jnp.dot(agg_a[...] * invd, wla[...],
                 preferred_element_type=jnp.float32)
         + jnp.dot(agg_b[...] * invd, wlb[...],
                   preferred_element_type=jnp.float32)
         + jnp.dot(h1a[...], wra[...], preferred_element_type=jnp.float32)
         + jnp.dot(h1b[...], wrb[...], preferred_element_type=jnp.float32)
         + b2[...])
    h = jnp.maximum(h, 0.0)
    logists[...] = jnp.dot(h, wcls[...],
                           preferred_element_type=jnp.float32) + bcls[...]
    pre = jnp.dot(h, wc[...], preferred_element_type=jnp.float32)
    out[...] = jnp.tanh(pre * bn_scale[...] + bn_shift[...])


def _row_spec(cols):
    return pl.BlockSpec((RB, cols), lambda i: (i, 0))


def _full_spec(r, c):
    return pl.BlockSpec((r, c), lambda i: (0, 0))


def kernel(features, edges, W1l, W1r, b1, W2l, W2r, b2, Wc, bc, Wcls, bcls,
           gamma, beta, run_mean, run_var):
    f32 = jnp.float32
    xa = features[:, :FH]
    xb = features[:, FH:]
    src = edges[0]
    dst = edges[1]
    zeros2d = jnp.zeros((ROWS_B, FH), f32)
    zeros1d = jnp.zeros((ROWS_B,), f32)

    agg1a, agg1b, deg = _sc_agg_deg(xa, xb, src, dst, zeros2d, zeros1d)
    deg2 = deg.reshape(N, 1)

    w1la = jnp.transpose(W1l[:, :FH])
    w1lb = jnp.transpose(W1l[:, FH:])
    w1ra = jnp.transpose(W1r[:, :FH])
    w1rb = jnp.transpose(W1r[:, FH:])
    HID = W1l.shape[0]
    h1a, h1b = pl.pallas_call(
        _tc_layer1_body,
        grid=(GRID,),
        in_specs=[_row_spec(FH), _row_spec(FH), _row_spec(FH), _row_spec(FH),
                  _row_spec(1),
                  _full_spec(FH, HID), _full_spec(FH, HID),
                  _full_spec(FH, HID), _full_spec(FH, HID),
                  _full_spec(1, HID)],
        out_specs=[_row_spec(FH), _row_spec(FH)],
        out_shape=[jax.ShapeDtypeStruct((N, FH), f32),
                   jax.ShapeDtypeStruct((N, FH), f32)],
    )(agg1a, agg1b, xa, xb, deg2, w1la, w1lb, w1ra, w1rb,
      b1.reshape(1, HID))

    agg2a, agg2b = _sc_agg(h1a, h1b, src, dst, zeros2d, zeros1d)

    OUT = W2l.shape[0]
    NCLS = Wcls.shape[0]
    NBITS = Wc.shape[0]
    w2la = jnp.transpose(W2l[:, :FH])
    w2lb = jnp.transpose(W2l[:, FH:])
    w2ra = jnp.transpose(W2r[:, :FH])
    w2rb = jnp.transpose(W2r[:, FH:])
    wclsT = jnp.transpose(Wcls)
    wcT = jnp.transpose(Wc)
    bn_scale = gamma / jnp.sqrt(run_var + 1e-5)
    bn_shift = (bc - run_mean) * bn_scale + beta
    logists, out = pl.pallas_call(
        _tc_layer2_body,
        grid=(GRID,),
        in_specs=[_row_spec(FH), _row_spec(FH), _row_spec(FH), _row_spec(FH),
                  _row_spec(1),
                  _full_spec(FH, OUT), _full_spec(FH, OUT),
                  _full_spec(FH, OUT), _full_spec(FH, OUT),
                  _full_spec(1, OUT),
                  _full_spec(OUT, NCLS), _full_spec(1, NCLS),
                  _full_spec(OUT, NBITS),
                  _full_spec(1, NBITS), _full_spec(1, NBITS)],
        out_specs=[_row_spec(NCLS), _row_spec(NBITS)],
        out_shape=[jax.ShapeDtypeStruct((N, NCLS), f32),
                   jax.ShapeDtypeStruct((N, NBITS), f32)],
    )(agg2a, agg2b, h1a, h1b, deg2, w2la, w2lb, w2ra, w2rb,
      b2.reshape(1, OUT), wclsT, bcls.reshape(1, NCLS), wcT,
      bn_scale.reshape(1, NBITS), bn_shift.reshape(1, NBITS))

    return (logists, out)


# speedup vs baseline: 2.9663x; 2.9663x over previous
import jax
import jax.numpy as jnp
from jax import lax
from jax.experimental import pallas as pl
from jax.experimental.pallas import tpu as pltpu
from jax.experimental.pallas import tpu_sc as plsc

N = 10000
E = 160000
FH = 128
NTILES = 16
EPT = E // NTILES
B = 64
NFULL = EPT // B
TAIL = EPT - NFULL * B
ROWS_A = 624
ROWS_B = N - 15 * ROWS_A
NCH_A, REM_A = divmod(ROWS_A, B)
NCH_B, REM_B = divmod(ROWS_B, B)
EPT_D = E // (2 * NTILES)
B_D = 40
NB_D = EPT_D // B_D


def _sc_agg_body(xa, xb, src_h, dst_h, zeros2d,
                 agg_a, agg_b,
                 sidx, didx, sidx_t, didx_t, rows, acc_sh, sem):
    c = lax.axis_index("c")
    s = lax.axis_index("s")
    base_rows = s * ROWS_A

    pltpu.sync_copy(zeros2d, rows)

    @pl.when(s < 15)
    def _():
        for j in range(NCH_A):
            pltpu.sync_copy(rows, acc_sh.at[pl.ds(base_rows + j * B, B)])
        if REM_A:
            pltpu.sync_copy(rows.at[pl.ds(0, REM_A)],
                            acc_sh.at[pl.ds(base_rows + NCH_A * B, REM_A)])

    @pl.when(s == 15)
    def _():
        for j in range(NCH_B):
            pltpu.sync_copy(rows, acc_sh.at[pl.ds(15 * ROWS_A + j * B, B)])
        if REM_B:
            pltpu.sync_copy(rows.at[pl.ds(0, REM_B)],
                            acc_sh.at[pl.ds(15 * ROWS_A + NCH_B * B, REM_B)])

    plsc.subcore_barrier()

    def edge_loop(table):
        def body(i, carry):
            base = pl.multiple_of(s * EPT + i * B, 8)
            pltpu.sync_copy(src_h.at[pl.ds(base, B)], sidx)
            pltpu.sync_copy(dst_h.at[pl.ds(base, B)], didx)
            pltpu.async_copy(table.at[sidx], rows, sem).wait()
            pltpu.sync_copy(rows, acc_sh.at[didx], add=True)
            return carry

        lax.fori_loop(0, NFULL, body, 0)
        tbase = pl.multiple_of(s * EPT + NFULL * B, 8)
        pltpu.sync_copy(src_h.at[pl.ds(tbase, TAIL)], sidx_t)
        pltpu.sync_copy(dst_h.at[pl.ds(tbase, TAIL)], didx_t)
        pltpu.async_copy(table.at[sidx_t], rows.at[pl.ds(0, TAIL)],
                         sem).wait()
        pltpu.sync_copy(rows.at[pl.ds(0, TAIL)], acc_sh.at[didx_t], add=True)

    @pl.when(c == 0)
    def _():
        edge_loop(xa)

    @pl.when(c == 1)
    def _():
        edge_loop(xb)

    plsc.subcore_barrier()

    def copy_chunk(dst_hbm, lo, n):
        pltpu.sync_copy(acc_sh.at[pl.ds(lo, n)], rows.at[pl.ds(0, n)])
        pltpu.sync_copy(rows.at[pl.ds(0, n)], dst_hbm.at[pl.ds(lo, n)])

    def copy_out(dst_hbm):
        @pl.when(s < 15)
        def _():
            for j in range(NCH_A):
                copy_chunk(dst_hbm, base_rows + j * B, B)
            if REM_A:
                copy_chunk(dst_hbm, base_rows + NCH_A * B, REM_A)

        @pl.when(s == 15)
        def _():
            for j in range(NCH_B):
                copy_chunk(dst_hbm, 15 * ROWS_A + j * B, B)
            if REM_B:
                copy_chunk(dst_hbm, 15 * ROWS_A + NCH_B * B, REM_B)

    @pl.when(c == 0)
    def _():
        copy_out(agg_a)

    @pl.when(c == 1)
    def _():
        copy_out(agg_b)


def _make_sc_agg():
    out_type = [jax.ShapeDtypeStruct((N, FH), jnp.float32),
                jax.ShapeDtypeStruct((N, FH), jnp.float32)]
    mesh = plsc.VectorSubcoreMesh(core_axis_name="c", subcore_axis_name="s")
    scratch = [
        pltpu.VMEM((B,), jnp.int32),
        pltpu.VMEM((B,), jnp.int32),
        pltpu.VMEM((TAIL,), jnp.int32),
        pltpu.VMEM((TAIL,), jnp.int32),
        pltpu.VMEM((B, FH), jnp.float32),
        pltpu.VMEM_SHARED((N, FH), jnp.float32),
        pltpu.SemaphoreType.DMA,
    ]
    return pl.kernel(_sc_agg_body, out_type=out_type, mesh=mesh,
                     scratch_types=scratch)


_sc_agg = _make_sc_agg()


def _sc_deg_body(dst_h, zeros2d, ones_h, deg_a, deg_b,
                 didx, ones_v, stage, deg_sh):
    c = lax.axis_index("c")
    s = lax.axis_index("s")
    base_rows = s * ROWS_A

    pltpu.sync_copy(zeros2d, stage)
    pltpu.sync_copy(ones_h, ones_v)

    @pl.when(s < 15)
    def _():
        for j in range(NCH_A):
            pltpu.sync_copy(stage.at[pl.ds(0, B)],
                            deg_sh.at[pl.ds(base_rows + j * B, B)])
        if REM_A:
            pltpu.sync_copy(stage.at[pl.ds(0, REM_A)],
                            deg_sh.at[pl.ds(base_rows + NCH_A * B, REM_A)])

    @pl.when(s == 15)
    def _():
        for j in range(NCH_B):
            pltpu.sync_copy(stage.at[pl.ds(0, B)],
                            deg_sh.at[pl.ds(15 * ROWS_A + j * B, B)])
        if REM_B:
            pltpu.sync_copy(stage.at[pl.ds(0, REM_B)],
                            deg_sh.at[pl.ds(15 * ROWS_A + NCH_B * B, REM_B)])

    plsc.subcore_barrier()

    def body(i, carry):
        base = pl.multiple_of((c * NTILES + s) * EPT_D + i * B_D, 8)
        pltpu.sync_copy(dst_h.at[pl.ds(base, B_D)], didx)
        pltpu.sync_copy(ones_v, deg_sh.at[didx], add=True)
        return carry

    lax.fori_loop(0, NB_D, body, 0)

    plsc.subcore_barrier()

    def copy_chunk(dst_hbm, lo, n):
        pltpu.sync_copy(deg_sh.at[pl.ds(lo, n)], stage.at[pl.ds(0, n)])
        pltpu.sync_copy(stage.at[pl.ds(0, n)], dst_hbm.at[pl.ds(lo, n)])

    def copy_out(dst_hbm):
        @pl.when(s < 15)
        def _():
            for j in range(NCH_A):
                copy_chunk(dst_hbm, base_rows + j * B, B)
            if REM_A:
                copy_chunk(dst_hbm, base_rows + NCH_A * B, REM_A)

        @pl.when(s == 15)
        def _():
            for j in range(NCH_B):
                copy_chunk(dst_hbm, 15 * ROWS_A + j * B, B)
            if REM_B:
                copy_chunk(dst_hbm, 15 * ROWS_A + NCH_B * B, REM_B)

    @pl.when(c == 0)
    def _():
        copy_out(deg_a)

    @pl.when(c == 1)
    def _():
        copy_out(deg_b)


def _make_sc_deg():
    out_type = [jax.ShapeDtypeStruct((N, FH), jnp.float32),
                jax.ShapeDtypeStruct((N, FH), jnp.float32)]
    mesh = plsc.VectorSubcoreMesh(core_axis_name="c", subcore_axis_name="s")
    scratch = [
        pltpu.VMEM((B_D,), jnp.int32),
        pltpu.VMEM((B_D, FH), jnp.float32),
        pltpu.VMEM((B, FH), jnp.float32),
        pltpu.VMEM_SHARED((N, FH), jnp.float32),
    ]
    return pl.kernel(_sc_deg_body, out_type=out_type, mesh=mesh,
                     scratch_types=scratch)


_sc_deg = _make_sc_deg()

RB = 1000
GRID = N // RB


def _tc_layer1_body(agg_a, agg_b, xa, xb, dega, degb, wla, wlb, wra, wrb,
                    b1, h1a, h1b):
    deg = dega[:, :1] + degb[:, :1]
    invd = 1.0 / jnp.maximum(deg, 1.0)
    h = (jnp.dot(agg_a[...] * invd, wla[...],
                 preferred_element_type=jnp.float32)
         + jnp.dot(agg_b[...] * invd, wlb[...],
                   preferred_element_type=jnp.float32)
         + jnp.dot(xa[...], wra[...], preferred_element_type=jnp.float32)
         + jnp.dot(xb[...], wrb[...], preferred_element_type=jnp.float32)
         + b1[...])
    h = jnp.maximum(h, 0.0)
    h1a[...] = h[:, :FH]
    h1b[...] = h[:, FH:]


def _tc_layer2_body(agg_a, agg_b, h1a, h1b, dega, degb, wla, wlb, wra, wrb,
                    b2, wcls, bcls, wc, bn_scale, bn_shift, logists, out):
    deg = dega[:, :1] + degb[:, :1]
    invd = 1.0 / jnp.maximum(deg, 1.0)
    h = (jnp.dot(agg_a[...] * invd, wla[...],
                 preferred_element_type=jnp.float32)
         + jnp.dot(agg_b[...] * invd, wlb[...],
                   preferred_element_type=jnp.float32)
         + jnp.dot(h1a[...], wra[...], preferred_element_type=jnp.float32)
         + jnp.dot(h1b[...], wrb[...], preferred_element_type=jnp.float32)
         + b2[...])
    h = jnp.maximum(h, 0.0)
    logists[...] = jnp.dot(h, wcls[...],
                           preferred_element_type=jnp.float32) + bcls[...]
    pre = jnp.dot(h, wc[...], preferred_element_type=jnp.float32)
    out[...] = jnp.tanh(pre * bn_scale[...] + bn_shift[...])


def _row_spec(cols):
    return pl.BlockSpec((RB, cols), lambda i: (i, 0))


def _full_spec(r, c):
    return pl.BlockSpec((r, c), lambda i: (0, 0))


def kernel(features, edges, W1l, W1r, b1, W2l, W2r, b2, Wc, bc, Wcls, bcls,
           gamma, beta, run_mean, run_var):
    f32 = jnp.float32
    xa = features[:, :FH]
    xb = features[:, FH:]
    src = edges[0]
    dst = edges[1]
    zeros2d = jnp.zeros((B, FH), f32)
    ones_h = jnp.ones((B_D, FH), f32)

    dega, degb = _sc_deg(dst, zeros2d, ones_h)
    agg1a, agg1b = _sc_agg(xa, xb, src, dst, zeros2d)

    w1la = jnp.transpose(W1l[:, :FH])
    w1lb = jnp.transpose(W1l[:, FH:])
    w1ra = jnp.transpose(W1r[:, :FH])
    w1rb = jnp.transpose(W1r[:, FH:])
    HID = W1l.shape[0]
    h1a, h1b = pl.pallas_call(
        _tc_layer1_body,
        grid=(GRID,),
        in_specs=[_row_spec(FH), _row_spec(FH), _row_spec(FH), _row_spec(FH),
                  _row_spec(FH), _row_spec(FH),
                  _full_spec(FH, HID), _full_spec(FH, HID),
                  _full_spec(FH, HID), _full_spec(FH, HID),
                  _full_spec(1, HID)],
        out_specs=[_row_spec(FH), _row_spec(FH)],
        out_shape=[jax.ShapeDtypeStruct((N, FH), f32),
                   jax.ShapeDtypeStruct((N, FH), f32)],
    )(agg1a, agg1b, xa, xb, dega, degb, w1la, w1lb, w1ra, w1rb,
      b1.reshape(1, HID))

    agg2a, agg2b = _sc_agg(h1a, h1b, src, dst, zeros2d)

    OUT = W2l.shape[0]
    NCLS = Wcls.shape[0]
    NBITS = Wc.shape[0]
    w2la = jnp.transpose(W2l[:, :FH])
    w2lb = jnp.transpose(W2l[:, FH:])
    w2ra = jnp.transpose(W2r[:, :FH])
    w2rb = jnp.transpose(W2r[:, FH:])
    wclsT = jnp.transpose(Wcls)
    wcT = jnp.transpose(Wc)
    bn_scale = gamma / jnp.sqrt(run_var + 1e-5)
    bn_shift = (bc - run_mean) * bn_scale + beta
    logists, out = pl.pallas_call(
        _tc_layer2_body,
        grid=(GRID,),
        in_specs=[_row_spec(FH), _row_spec(FH), _row_spec(FH), _row_spec(FH),
                  _row_spec(FH), _row_spec(FH),
                  _full_spec(FH, OUT), _full_spec(FH, OUT),
                  _full_spec(FH, OUT), _full_spec(FH, OUT),
                  _full_spec(1, OUT),
                  _full_spec(OUT, NCLS), _full_spec(1, NCLS),
                  _full_spec(OUT, NBITS),
                  _full_spec(1, NBITS), _full_spec(1, NBITS)],
        out_specs=[_row_spec(NCLS), _row_spec(NBITS)],
        out_shape=[jax.ShapeDtypeStruct((N, NCLS), f32),
                   jax.ShapeDtypeStruct((N, NBITS), f32)],
    )(agg2a, agg2b, h1a, h1b, dega, degb, w2la, w2lb, w2ra, w2rb,
      b2.reshape(1, OUT), wclsT, bcls.reshape(1, NCLS), wcT,
      bn_scale.reshape(1, NBITS), bn_shift.reshape(1, NBITS))

    return (logists, out)


# speedup vs baseline: 4.4000x; 1.4833x over previous
import jax
import jax.numpy as jnp
from jax import lax
from jax.experimental import pallas as pl
from jax.experimental.pallas import tpu as pltpu
from jax.experimental.pallas import tpu_sc as plsc

N = 10000
E = 160000
FH = 128
NTILES = 16
EPT = E // NTILES
B = 64
NFULL = EPT // B
TAIL = EPT - NFULL * B
ROWS_A = 624
ROWS_B = N - 15 * ROWS_A
NCH_A, REM_A = divmod(ROWS_A, B)
NCH_B, REM_B = divmod(ROWS_B, B)
EPT_D = E // (2 * NTILES)
B_D = 40
NB_D = EPT_D // B_D


def _sc_agg_body(xa, xb, src_h, dst_h, zeros2d,
                 agg_a, agg_b,
                 sidx, didx, sidx_t, didx_t, rows, sidx1, didx1, rows1,
                 acc_sh, sem, sem1):
    c = lax.axis_index("c")
    s = lax.axis_index("s")
    base_rows = s * ROWS_A

    pltpu.sync_copy(zeros2d, rows)

    @pl.when(s < 15)
    def _():
        for j in range(NCH_A):
            pltpu.sync_copy(rows, acc_sh.at[pl.ds(base_rows + j * B, B)])
        if REM_A:
            pltpu.sync_copy(rows.at[pl.ds(0, REM_A)],
                            acc_sh.at[pl.ds(base_rows + NCH_A * B, REM_A)])

    @pl.when(s == 15)
    def _():
        for j in range(NCH_B):
            pltpu.sync_copy(rows, acc_sh.at[pl.ds(15 * ROWS_A + j * B, B)])
        if REM_B:
            pltpu.sync_copy(rows.at[pl.ds(0, REM_B)],
                            acc_sh.at[pl.ds(15 * ROWS_A + NCH_B * B, REM_B)])

    plsc.subcore_barrier()

    def edge_loop(table):
        bufs = ((sidx, didx, rows, sem), (sidx1, didx1, rows1, sem1))

        def load_and_fire(b, k):
            si, di, rw, sm = bufs[k]
            base = pl.multiple_of(s * EPT + b * B, 8)
            pltpu.sync_copy(src_h.at[pl.ds(base, B)], si)
            pltpu.sync_copy(dst_h.at[pl.ds(base, B)], di)
            pltpu.async_copy(table.at[si], rw, sm)

        def wait_and_scatter(k):
            si, di, rw, sm = bufs[k]
            pltpu.make_async_copy(table.at[si], rw, sm).wait()
            pltpu.sync_copy(rw, acc_sh.at[di], add=True)

        load_and_fire(0, 0)

        def body(i, carry):
            t = i * 2
            load_and_fire(t + 1, 1)
            wait_and_scatter(0)
            @pl.when(i < NFULL // 2 - 1)
            def _():
                load_and_fire(t + 2, 0)
            wait_and_scatter(1)
            return carry

        lax.fori_loop(0, NFULL // 2, body, 0)
        tbase = pl.multiple_of(s * EPT + NFULL * B, 8)
        pltpu.sync_copy(src_h.at[pl.ds(tbase, TAIL)], sidx_t)
        pltpu.sync_copy(dst_h.at[pl.ds(tbase, TAIL)], didx_t)
        pltpu.async_copy(table.at[sidx_t], rows.at[pl.ds(0, TAIL)],
                         sem).wait()
        pltpu.sync_copy(rows.at[pl.ds(0, TAIL)], acc_sh.at[didx_t], add=True)

    @pl.when(c == 0)
    def _():
        edge_loop(xa)

    @pl.when(c == 1)
    def _():
        edge_loop(xb)

    plsc.subcore_barrier()

    def copy_chunk(dst_hbm, lo, n):
        pltpu.sync_copy(acc_sh.at[pl.ds(lo, n)], rows.at[pl.ds(0, n)])
        pltpu.sync_copy(rows.at[pl.ds(0, n)], dst_hbm.at[pl.ds(lo, n)])

    def copy_out(dst_hbm):
        @pl.when(s < 15)
        def _():
            for j in range(NCH_A):
                copy_chunk(dst_hbm, base_rows + j * B, B)
            if REM_A:
                copy_chunk(dst_hbm, base_rows + NCH_A * B, REM_A)

        @pl.when(s == 15)
        def _():
            for j in range(NCH_B):
                copy_chunk(dst_hbm, 15 * ROWS_A + j * B, B)
            if REM_B:
                copy_chunk(dst_hbm, 15 * ROWS_A + NCH_B * B, REM_B)

    @pl.when(c == 0)
    def _():
        copy_out(agg_a)

    @pl.when(c == 1)
    def _():
        copy_out(agg_b)


def _make_sc_agg():
    out_type = [jax.ShapeDtypeStruct((N, FH), jnp.float32),
                jax.ShapeDtypeStruct((N, FH), jnp.float32)]
    mesh = plsc.VectorSubcoreMesh(core_axis_name="c", subcore_axis_name="s")
    scratch = [
        pltpu.VMEM((B,), jnp.int32),
        pltpu.VMEM((B,), jnp.int32),
        pltpu.VMEM((TAIL,), jnp.int32),
        pltpu.VMEM((TAIL,), jnp.int32),
        pltpu.VMEM((B, FH), jnp.float32),
        pltpu.VMEM((B,), jnp.int32),
        pltpu.VMEM((B,), jnp.int32),
        pltpu.VMEM((B, FH), jnp.float32),
        pltpu.VMEM_SHARED((N, FH), jnp.float32),
        pltpu.SemaphoreType.DMA,
        pltpu.SemaphoreType.DMA,
    ]
    return pl.kernel(_sc_agg_body, out_type=out_type, mesh=mesh,
                     scratch_types=scratch)


_sc_agg = _make_sc_agg()


def _sc_deg_body(dst_h, zeros2d, ones_h, deg_a, deg_b,
                 didx, ones_v, stage, deg_sh):
    c = lax.axis_index("c")
    s = lax.axis_index("s")
    base_rows = s * ROWS_A

    pltpu.sync_copy(zeros2d, stage)
    pltpu.sync_copy(ones_h, ones_v)

    @pl.when(s < 15)
    def _():
        for j in range(NCH_A):
            pltpu.sync_copy(stage.at[pl.ds(0, B)],
                            deg_sh.at[pl.ds(base_rows + j * B, B)])
        if REM_A:
            pltpu.sync_copy(stage.at[pl.ds(0, REM_A)],
                            deg_sh.at[pl.ds(base_rows + NCH_A * B, REM_A)])

    @pl.when(s == 15)
    def _():
        for j in range(NCH_B):
            pltpu.sync_copy(stage.at[pl.ds(0, B)],
                            deg_sh.at[pl.ds(15 * ROWS_A + j * B, B)])
        if REM_B:
            pltpu.sync_copy(stage.at[pl.ds(0, REM_B)],
                            deg_sh.at[pl.ds(15 * ROWS_A + NCH_B * B, REM_B)])

    plsc.subcore_barrier()

    def body(i, carry):
        base = pl.multiple_of((c * NTILES + s) * EPT_D + i * B_D, 8)
        pltpu.sync_copy(dst_h.at[pl.ds(base, B_D)], didx)
        pltpu.sync_copy(ones_v, deg_sh.at[didx], add=True)
        return carry

    lax.fori_loop(0, NB_D, body, 0)

    plsc.subcore_barrier()

    def copy_chunk(dst_hbm, lo, n):
        pltpu.sync_copy(deg_sh.at[pl.ds(lo, n)], stage.at[pl.ds(0, n)])
        pltpu.sync_copy(stage.at[pl.ds(0, n)], dst_hbm.at[pl.ds(lo, n)])

    def copy_out(dst_hbm):
        @pl.when(s < 15)
        def _():
            for j in range(NCH_A):
                copy_chunk(dst_hbm, base_rows + j * B, B)
            if REM_A:
                copy_chunk(dst_hbm, base_rows + NCH_A * B, REM_A)

        @pl.when(s == 15)
        def _():
            for j in range(NCH_B):
                copy_chunk(dst_hbm, 15 * ROWS_A + j * B, B)
            if REM_B:
                copy_chunk(dst_hbm, 15 * ROWS_A + NCH_B * B, REM_B)

    @pl.when(c == 0)
    def _():
        copy_out(deg_a)

    @pl.when(c == 1)
    def _():
        copy_out(deg_b)


def _make_sc_deg():
    out_type = [jax.ShapeDtypeStruct((N, FH), jnp.float32),
                jax.ShapeDtypeStruct((N, FH), jnp.float32)]
    mesh = plsc.VectorSubcoreMesh(core_axis_name="c", subcore_axis_name="s")
    scratch = [
        pltpu.VMEM((B_D,), jnp.int32),
        pltpu.VMEM((B_D, FH), jnp.float32),
        pltpu.VMEM((B, FH), jnp.float32),
        pltpu.VMEM_SHARED((N, FH), jnp.float32),
    ]
    return pl.kernel(_sc_deg_body, out_type=out_type, mesh=mesh,
                     scratch_types=scratch)


_sc_deg = _make_sc_deg()

RB = 1000
GRID = N // RB


def _tc_layer1_body(agg_a, agg_b, xa, xb, dega, degb, wla, wlb, wra, wrb,
                    b1, h1a, h1b):
    deg = dega[:, :1] + degb[:, :1]
    invd = 1.0 / jnp.maximum(deg, 1.0)
    h = (jnp.dot(agg_a[...] * invd, wla[...],
                 preferred_element_type=jnp.float32)
         + jnp.dot(agg_b[...] * invd, wlb[...],
                   preferred_element_type=jnp.float32)
         + jnp.dot(xa[...], wra[...], preferred_element_type=jnp.float32)
         + jnp.dot(xb[...], wrb[...], preferred_element_type=jnp.float32)
         + b1[...])
    h = jnp.maximum(h, 0.0)
    h1a[...] = h[:, :FH]
    h1b[...] = h[:, FH:]


def _tc_layer2_body(agg_a, agg_b, h1a, h1b, dega, degb, wla, wlb, wra, wrb,
                    b2, wcls, bcls, wc, bn_scale, bn_shift, logists, out):
    deg = dega[:, :1] + degb[:, :1]
    invd = 1.0 / jnp.maximum(deg, 1.0)
    h = (jnp.dot(agg_a[...] * invd, wla[...],
                 preferred_element_type=jnp.float32)
         + jnp.dot(agg_b[...] * invd, wlb[...],
                   preferred_element_type=jnp.float32)
         + jnp.dot(h1a[...], wra[...], preferred_element_type=jnp.float32)
         + jnp.dot(h1b[...], wrb[...], preferred_element_type=jnp.float32)
         + b2[...])
    h = jnp.maximum(h, 0.0)
    logists[...] = jnp.dot(h, wcls[...],
                           preferred_element_type=jnp.float32) + bcls[...]
    pre = jnp.dot(h, wc[...], preferred_element_type=jnp.float32)
    out[...] = jnp.tanh(pre * bn_scale[...] + bn_shift[...])


def _row_spec(cols):
    return pl.BlockSpec((RB, cols), lambda i: (i, 0))


def _full_spec(r, c):
    return pl.BlockSpec((r, c), lambda i: (0, 0))


def kernel(features, edges, W1l, W1r, b1, W2l, W2r, b2, Wc, bc, Wcls, bcls,
           gamma, beta, run_mean, run_var):
    f32 = jnp.float32
    xa = features[:, :FH]
    xb = features[:, FH:]
    src = edges[0]
    dst = edges[1]
    zeros2d = jnp.zeros((B, FH), f32)
    ones_h = jnp.ones((B_D, FH), f32)

    dega, degb = _sc_deg(dst, zeros2d, ones_h)
    agg1a, agg1b = _sc_agg(xa, xb, src, dst, zeros2d)

    w1la = jnp.transpose(W1l[:, :FH])
    w1lb = jnp.transpose(W1l[:, FH:])
    w1ra = jnp.transpose(W1r[:, :FH])
    w1rb = jnp.transpose(W1r[:, FH:])
    HID = W1l.shape[0]
    h1a, h1b = pl.pallas_call(
        _tc_layer1_body,
        grid=(GRID,),
        in_specs=[_row_spec(FH), _row_spec(FH), _row_spec(FH), _row_spec(FH),
                  _row_spec(FH), _row_spec(FH),
                  _full_spec(FH, HID), _full_spec(FH, HID),
                  _full_spec(FH, HID), _full_spec(FH, HID),
                  _full_spec(1, HID)],
        out_specs=[_row_spec(FH), _row_spec(FH)],
        out_shape=[jax.ShapeDtypeStruct((N, FH), f32),
                   jax.ShapeDtypeStruct((N, FH), f32)],
    )(agg1a, agg1b, xa, xb, dega, degb, w1la, w1lb, w1ra, w1rb,
      b1.reshape(1, HID))

    agg2a, agg2b = _sc_agg(h1a, h1b, src, dst, zeros2d)

    OUT = W2l.shape[0]
    NCLS = Wcls.shape[0]
    NBITS = Wc.shape[0]
    w2la = jnp.transpose(W2l[:, :FH])
    w2lb = jnp.transpose(W2l[:, FH:])
    w2ra = jnp.transpose(W2r[:, :FH])
    w2rb = jnp.transpose(W2r[:, FH:])
    wclsT = jnp.transpose(Wcls)
    wcT = jnp.transpose(Wc)
    bn_scale = gamma / jnp.sqrt(run_var + 1e-5)
    bn_shift = (bc - run_mean) * bn_scale + beta
    logists, out = pl.pallas_call(
        _tc_layer2_body,
        grid=(GRID,),
        in_specs=[_row_spec(FH), _row_spec(FH), _row_spec(FH), _row_spec(FH),
                  _row_spec(FH), _row_spec(FH),
                  _full_spec(FH, OUT), _full_spec(FH, OUT),
                  _full_spec(FH, OUT), _full_spec(FH, OUT),
                  _full_spec(1, OUT),
                  _full_spec(OUT, NCLS), _full_spec(1, NCLS),
                  _full_spec(OUT, NBITS),
                  _full_spec(1, NBITS), _full_spec(1, NBITS)],
        out_specs=[_row_spec(NCLS), _row_spec(NBITS)],
        out_shape=[jax.ShapeDtypeStruct((N, NCLS), f32),
                   jax.ShapeDtypeStruct((N, NBITS), f32)],
    )(agg2a, agg2b, h1a, h1b, dega, degb, w2la, w2lb, w2ra, w2rb,
      b2.reshape(1, OUT), wclsT, bcls.reshape(1, NCLS), wcT,
      bn_scale.reshape(1, NBITS), bn_shift.reshape(1, NBITS))

    return (logists, out)
